# baseline (speedup 1.0000x reference)
import functools

import jax
import jax.numpy as jnp
from jax import lax
from jax.experimental import pallas as pl
from jax.experimental.pallas import tpu as pltpu
from jax.experimental.pallas import tpu_sc as plsc

B = 16384
D = 32
NNEG = 20
NC = 2
NS = 16
NW = NC * NS
RPW = B // NW
CH = 128
NCH = RPW // CH
CHN = CH * NNEG
NBLK = CH // 16

_mesh = plsc.VectorSubcoreMesh(core_axis_name="c", subcore_axis_name="s")


@functools.partial(
    pl.kernel,
    mesh=_mesh,
    compiler_params=pltpu.CompilerParams(
        needs_layout_passes=False, use_tc_tiling_on_sc=False
    ),
    out_type=(
        jax.ShapeDtypeStruct((B,), jnp.float32),
        jax.ShapeDtypeStruct((B,), jnp.float32),
    ),
    scratch_types=[
        pltpu.VMEM((RPW,), jnp.int32),
        pltpu.VMEM((RPW,), jnp.int32),
        pltpu.VMEM((RPW * NNEG,), jnp.int32),
        pltpu.VMEM((RPW, D), jnp.float32),
        pltpu.VMEM((RPW, D), jnp.float32),
        pltpu.VMEM((CHN, D), jnp.float32),
        pltpu.VMEM((RPW,), jnp.float32),
        pltpu.VMEM((RPW,), jnp.float32),
        pltpu.SemaphoreType.DMA,
        pltpu.SemaphoreType.DMA,
    ],
)
def _sc_scores(center_hbm, context_hbm, negflat_hbm, in_hbm, out_hbm,
               pos_hbm, negsum_hbm,
               cidx, tidx, nidx, crow, trow, nrow, posb, negb, sem0, sem1):
    wid = lax.axis_index("s") * NC + lax.axis_index("c")
    base = wid * RPW

    pltpu.sync_copy(center_hbm.at[pl.ds(base, RPW)], cidx)
    pltpu.sync_copy(context_hbm.at[pl.ds(base, RPW)], tidx)
    pltpu.sync_copy(negflat_hbm.at[pl.ds(base * NNEG, RPW * NNEG)], nidx)

    cp0 = pltpu.async_copy(in_hbm.at[cidx], crow, sem0)
    cp1 = pltpu.async_copy(out_hbm.at[tidx], trow, sem1)
    cp0.wait()
    cp1.wait()

    lane = lax.iota(jnp.int32, 16)

    def chunk_body(ch, _):
        pltpu.async_copy(
            out_hbm.at[nidx.at[pl.ds(ch * CHN, CHN)]], nrow, sem0
        ).wait()

        def blk_body(blk, _):
            crow_idx = blk * 16 + lane
            grow_idx = ch * CH + crow_idx
            pair0 = crow_idx * NNEG
            accp = jnp.zeros((16,), jnp.float32)
            accn = [jnp.zeros((16,), jnp.float32) for _ in range(NNEG)]
            for d in range(D):
                dsp = jnp.full((16,), d, jnp.int32)
                cg = plsc.load_gather(crow, [grow_idx, dsp])
                tg = plsc.load_gather(trow, [grow_idx, dsp])
                accp = accp + cg * tg
                for n in range(NNEG):
                    gn = plsc.load_gather(nrow, [pair0 + n, dsp])
                    accn[n] = accn[n] + gn * cg
            posv = 1.0 / (1.0 + jnp.exp(-accp))
            negv = jnp.zeros((16,), jnp.float32)
            for n in range(NNEG):
                negv = negv + 1.0 / (1.0 + jnp.exp(accn[n]))
            r0 = ch * CH + blk * 16
            posb[pl.ds(r0, 16)] = posv
            negb[pl.ds(r0, 16)] = negv
            return 0

        lax.fori_loop(0, NBLK, blk_body, 0)
        return 0

    lax.fori_loop(0, NCH, chunk_body, 0)

    pltpu.sync_copy(posb, pos_hbm.at[pl.ds(base, RPW)])
    pltpu.sync_copy(negb, negsum_hbm.at[pl.ds(base, RPW)])


def _loss_body(pos_ref, neg_ref, out_ref):
    total = jnp.sum(jnp.log(pos_ref[...])) + jnp.sum(jnp.log(neg_ref[...]))
    out_ref[0, 0] = -total / B


_finish = pl.pallas_call(
    _loss_body,
    out_shape=jax.ShapeDtypeStruct((1, 1), jnp.float32),
    out_specs=pl.BlockSpec(memory_space=pltpu.SMEM),
)


def kernel(center, context, negative, in_embed, out_embed):
    negflat = negative.reshape(-1)
    pos, neg = _sc_scores(center, context, negflat, in_embed, out_embed)
    loss = _finish(pos.reshape(128, 128), neg.reshape(128, 128))
    return loss[0, 0]
